# baseline (speedup 1.0000x reference)
import functools

import jax
import jax.numpy as jnp
from jax import lax
from jax.experimental import pallas as pl
from jax.experimental.pallas import tpu as pltpu
from jax.experimental.pallas import tpu_sc as plsc

NUM_STOCKS = 100000
CELL_SIZE = 64
BATCH = 16384

NC, NS = 2, 16
NW = NC * NS
B_PER_W = BATCH // NW


HALF = CELL_SIZE // 2


def _gather_half(idx_flat, emb_half):
    mesh = plsc.VectorSubcoreMesh(core_axis_name="c", subcore_axis_name="s")
    out_t = jax.ShapeDtypeStruct((BATCH, HALF), jnp.float32)

    @functools.partial(
        pl.kernel,
        out_type=out_t,
        mesh=mesh,
        scratch_types=[
            pltpu.VMEM((B_PER_W,), jnp.int32),
            pltpu.VMEM((B_PER_W, HALF), jnp.float32),
            pltpu.SemaphoreType.DMA,
            pltpu.SemaphoreType.DMA,
        ],
    )
    def k(e_hbm, idx_hbm, o_hbm, idx_v, rows_v, sem_g, sem_w):
        wid = lax.axis_index("s") * NC + lax.axis_index("c")
        base = wid * B_PER_W
        pltpu.sync_copy(idx_hbm.at[pl.ds(base, B_PER_W)], idx_v)

        @pl.loop(0, B_PER_W, step=16)
        def _(j):
            v = idx_v[pl.ds(j, 16)]
            for t in range(16):
                pltpu.make_async_copy(
                    e_hbm.at[v[t]], rows_v.at[j + t], sem_g).start()

        pltpu.make_async_copy(
            o_hbm.at[pl.ds(base, B_PER_W)], rows_v, sem_g).wait()
        pltpu.async_copy(
            rows_v, o_hbm.at[pl.ds(base, B_PER_W)], sem_w).wait()

    return k(emb_half, idx_flat)


def kernel(Stock_ID, emb0, emb1):
    idx_flat = Stock_ID.reshape(BATCH).astype(jnp.int32)
    halves = [emb0[:, :HALF], emb0[:, HALF:], emb1[:, :HALF], emb1[:, HALF:]]
    outs = [_gather_half(idx_flat, h) for h in halves]
    o0 = jnp.concatenate(outs[:2], axis=1)
    o1 = jnp.concatenate(outs[2:], axis=1)
    return (o0, o1)


# speedup vs baseline: 1.9876x; 1.9876x over previous
import functools

import jax
import jax.numpy as jnp
from jax import lax
from jax.experimental import pallas as pl
from jax.experimental.pallas import tpu as pltpu
from jax.experimental.pallas import tpu_sc as plsc

NUM_STOCKS = 100000
CELL_SIZE = 64
BATCH = 16384

NC, NS = 2, 16
NW = NC * NS
B_PER_W = BATCH // NW


def _gather_one(idx_flat, emb):
    mesh = plsc.VectorSubcoreMesh(core_axis_name="c", subcore_axis_name="s")
    out_t = jax.ShapeDtypeStruct((BATCH, CELL_SIZE), jnp.float32)

    @functools.partial(
        pl.kernel,
        out_type=out_t,
        mesh=mesh,
        scratch_types=[
            pltpu.VMEM((B_PER_W,), jnp.int32),
            pltpu.VMEM((B_PER_W, CELL_SIZE), jnp.float32),
            pltpu.SemaphoreType.DMA,
            pltpu.SemaphoreType.DMA,
        ],
    )
    def k(e_hbm, idx_hbm, o_hbm, idx_v, rows_v, sem_g, sem_w):
        wid = lax.axis_index("s") * NC + lax.axis_index("c")
        base = wid * B_PER_W
        pltpu.sync_copy(idx_hbm.at[pl.ds(base, B_PER_W)], idx_v)

        @pl.loop(0, B_PER_W, step=16)
        def _(j):
            v = idx_v[pl.ds(j, 16)]
            for t in range(16):
                pltpu.make_async_copy(
                    e_hbm.at[v[t]], rows_v.at[j + t], sem_g).start()

        pltpu.make_async_copy(
            o_hbm.at[pl.ds(base, B_PER_W)], rows_v, sem_g).wait()
        pltpu.async_copy(
            rows_v, o_hbm.at[pl.ds(base, B_PER_W)], sem_w).wait()

    return k(emb, idx_flat)


def kernel(Stock_ID, emb0, emb1):
    idx_flat = Stock_ID.reshape(BATCH).astype(jnp.int32)
    o0 = _gather_one(idx_flat, emb0)
    o1 = _gather_one(idx_flat, emb1)
    return (o0, o1)
